# Initial kernel scaffold; baseline (speedup 1.0000x reference)
import jax
import jax.numpy as jnp
from jax.experimental import pallas as pl


def kernel(nodes, node_map, neigh_idx, features, W, gamma, beta):
    raise NotImplementedError("write your pallas kernel here")


# speedup vs baseline: 1.7560x; 1.7560x over previous
import functools

import jax
import jax.numpy as jnp
from jax import lax
from jax.experimental import pallas as pl
from jax.experimental.pallas import tpu as pltpu
from jax.experimental.pallas import tpu_sc as plsc

B = 8192
D = 256
K = 10
N = 50000
NC = 2
NS = 16
NW = NC * NS
BPW = B // NW
C = 8
NCH = BPW // C
SC = 64
NSC = BPW // SC


def _sc_gather_body(nodes_h, nmap_h, nidx_h, feat_h, self_o, sum_o,
                    nid_v, map_v, nbr_v, selfbuf, nbuf, sumbuf, sem):
    wid = lax.axis_index("s") * NC + lax.axis_index("c")
    base = wid * BPW

    pltpu.sync_copy(nodes_h.at[pl.ds(base, BPW)], nid_v)
    pltpu.sync_copy(nidx_h.at[wid], nbr_v)

    pltpu.async_copy(nmap_h.at[nid_v], map_v, sem).wait()

    for s in range(NSC):
        pltpu.async_copy(feat_h.at[map_v.at[pl.ds(s * SC, SC)]], selfbuf,
                         sem).wait()
        pltpu.sync_copy(selfbuf, self_o.at[pl.ds(base + s * SC, SC)])

    def chunk_body(c, carry):
        pltpu.async_copy(feat_h.at[nbr_v.at[c]], nbuf, sem).wait()

        def node_body(r, carry2):
            rb = r * K
            for d in range(D // 16):
                acc = nbuf[rb, pl.ds(d * 16, 16)]
                for j in range(1, K):
                    acc = acc + nbuf[rb + j, pl.ds(d * 16, 16)]
                sumbuf[r, pl.ds(d * 16, 16)] = acc
            return carry2

        lax.fori_loop(0, C, node_body, 0)
        pltpu.sync_copy(sumbuf, sum_o.at[pl.ds(base + c * C, C)])
        return carry

    lax.fori_loop(0, NCH, chunk_body, 0)


@functools.cache
def _make_sc_gather():
    return pl.kernel(
        _sc_gather_body,
        mesh=plsc.VectorSubcoreMesh(core_axis_name="c", subcore_axis_name="s"),
        out_type=[
            jax.ShapeDtypeStruct((B, D), jnp.float32),
            jax.ShapeDtypeStruct((B, D), jnp.float32),
        ],
        scratch_types=[
            pltpu.VMEM((BPW,), jnp.int32),
            pltpu.VMEM((BPW,), jnp.int32),
            pltpu.VMEM((NCH, C * K), jnp.int32),
            pltpu.VMEM((SC, D), jnp.float32),
            pltpu.VMEM((C * K, D), jnp.float32),
            pltpu.VMEM((C, D), jnp.float32),
            pltpu.SemaphoreType.DMA,
        ],
    )


def _tc_body(self_ref, sum_ref, w_ref, g_ref, b_ref, out_ref):
    ws = w_ref[:, :D]
    wn = w_ref[:, D:]
    dn = (((1,), (1,)), ((), ()))
    h = lax.dot_general(ws, self_ref[...], dn, preferred_element_type=jnp.float32)
    h = h + 0.1 * lax.dot_general(wn, sum_ref[...], dn,
                                  preferred_element_type=jnp.float32)
    mean = jnp.mean(h, axis=1, keepdims=True)
    cent = h - mean
    var = jnp.mean(cent * cent, axis=1, keepdims=True)
    inv = lax.rsqrt(var + 1e-5)
    out_ref[...] = jnp.maximum(cent * inv * g_ref[...] + b_ref[...], 0.0)


def _tc_fused(self_feats, neigh_sum, W, gamma2, beta2):
    return pl.pallas_call(
        _tc_body,
        out_shape=jax.ShapeDtypeStruct((D, B), jnp.float32),
    )(self_feats, neigh_sum, W, gamma2, beta2)


def kernel(nodes, node_map, neigh_idx, features, W, gamma, beta):
    nidx = neigh_idx.reshape(NW, NCH, C * K)
    self_feats, neigh_sum = _make_sc_gather()(nodes, node_map, nidx, features)
    return _tc_fused(self_feats, neigh_sum, W,
                     gamma.reshape(D, 1), beta.reshape(D, 1))


# speedup vs baseline: 2.5926x; 1.4764x over previous
import functools

import jax
import jax.numpy as jnp
from jax import lax
from jax.experimental import pallas as pl
from jax.experimental.pallas import tpu as pltpu
from jax.experimental.pallas import tpu_sc as plsc

B = 8192
D = 256
K = 10
N = 50000
NC = 2
NS = 16
NW = NC * NS
BPW = B // NW
C = 8
NCH = BPW // C
SC = 64
NSC = BPW // SC


def _reduce_chunk(nbuf, sumbuf):
    def node_body(r, carry):
        rb = r * K
        for d in range(D // 16):
            acc = nbuf[rb, pl.ds(d * 16, 16)]
            for j in range(1, K):
                acc = acc + nbuf[rb + j, pl.ds(d * 16, 16)]
            sumbuf[r, pl.ds(d * 16, 16)] = acc
        return carry
    lax.fori_loop(0, C, node_body, 0)


def _sc_gather_body(nodes_h, nmap_h, nidx_h, feat_h, self_o, sum_o,
                    nid_v, map_v, nbr_v, sbuf0, sbuf1, nbuf0, nbuf1,
                    qbuf0, qbuf1, semi, sem_s0, sem_s1, sem_n0, sem_n1,
                    sem_o0, sem_o1):
    wid = lax.axis_index("s") * NC + lax.axis_index("c")
    base = wid * BPW

    cp_nbr = pltpu.async_copy(nidx_h.at[wid], nbr_v, semi)
    pltpu.sync_copy(nodes_h.at[pl.ds(base, BPW)], nid_v)
    pltpu.async_copy(nmap_h.at[nid_v], map_v, sem_s0).wait()
    cp_nbr.wait()

    pltpu.async_copy(feat_h.at[nbr_v.at[0]], nbuf0, sem_n0)
    pltpu.async_copy(feat_h.at[nbr_v.at[1]], nbuf1, sem_n1)

    sbufs, ssems = (sbuf0, sbuf1), (sem_s0, sem_s1)
    gathers = []
    for s in range(2):
        gathers.append(pltpu.async_copy(
            feat_h.at[map_v.at[pl.ds(s * SC, SC)]], sbufs[s], ssems[s]))
    outs = [None, None]
    for s in range(NSC):
        b = s % 2
        gathers[s].wait()
        if outs[b] is not None:
            outs[b].wait()
        outs[b] = pltpu.async_copy(
            sbufs[b], self_o.at[pl.ds(base + s * SC, SC)], ssems[b])
        if s + 2 < NSC:
            if outs[b] is not None:
                outs[b].wait()
                outs[b] = None
            gathers.append(pltpu.async_copy(
                feat_h.at[map_v.at[pl.ds((s + 2) * SC, SC)]],
                sbufs[b], ssems[b]))
    for o in outs:
        o.wait()

    nbufs, qbufs = (nbuf0, nbuf1), (qbuf0, qbuf1)
    nsems, osems = (sem_n0, sem_n1), (sem_o0, sem_o1)

    def pair_body(p, carry):
        cc = p * 2
        for b in range(2):
            c = cc + b
            pltpu.make_async_copy(feat_h.at[nbr_v.at[c]], nbufs[b],
                                  nsems[b]).wait()
            @pl.when(p > 0)
            def _wait_prev():
                pltpu.make_async_copy(
                    qbufs[b], sum_o.at[pl.ds(base + (c - 2) * C, C)],
                    osems[b]).wait()
            _reduce_chunk(nbufs[b], qbufs[b])

            @pl.when(c + 2 < NCH)
            def _next_gather():
                pltpu.async_copy(feat_h.at[nbr_v.at[c + 2]], nbufs[b],
                                 nsems[b])
            pltpu.async_copy(qbufs[b], sum_o.at[pl.ds(base + c * C, C)],
                             osems[b])
        return carry

    lax.fori_loop(0, NCH // 2, pair_body, 0)
    for b in range(2):
        pltpu.make_async_copy(
            qbufs[b], sum_o.at[pl.ds(base + (NCH - 2 + b) * C, C)],
            osems[b]).wait()


@functools.cache
def _make_sc_gather():
    return pl.kernel(
        _sc_gather_body,
        mesh=plsc.VectorSubcoreMesh(core_axis_name="c", subcore_axis_name="s"),
        out_type=[
            jax.ShapeDtypeStruct((B, D), jnp.float32),
            jax.ShapeDtypeStruct((B, D), jnp.float32),
        ],
        scratch_types=[
            pltpu.VMEM((BPW,), jnp.int32),
            pltpu.VMEM((BPW,), jnp.int32),
            pltpu.VMEM((NCH, C * K), jnp.int32),
            pltpu.VMEM((SC, D), jnp.float32),
            pltpu.VMEM((SC, D), jnp.float32),
            pltpu.VMEM((C * K, D), jnp.float32),
            pltpu.VMEM((C * K, D), jnp.float32),
            pltpu.VMEM((C, D), jnp.float32),
            pltpu.VMEM((C, D), jnp.float32),
            pltpu.SemaphoreType.DMA,
            pltpu.SemaphoreType.DMA,
            pltpu.SemaphoreType.DMA,
            pltpu.SemaphoreType.DMA,
            pltpu.SemaphoreType.DMA,
            pltpu.SemaphoreType.DMA,
            pltpu.SemaphoreType.DMA,
        ],
    )


def _tc_body(self_ref, sum_ref, w_ref, g_ref, b_ref, out_ref):
    ws = w_ref[:, :D]
    wn = w_ref[:, D:]
    dn = (((1,), (1,)), ((), ()))
    h = lax.dot_general(ws, self_ref[...], dn, preferred_element_type=jnp.float32)
    h = h + 0.1 * lax.dot_general(wn, sum_ref[...], dn,
                                  preferred_element_type=jnp.float32)
    mean = jnp.mean(h, axis=1, keepdims=True)
    cent = h - mean
    var = jnp.mean(cent * cent, axis=1, keepdims=True)
    inv = lax.rsqrt(var + 1e-5)
    out_ref[...] = jnp.maximum(cent * inv * g_ref[...] + b_ref[...], 0.0)


def _tc_fused(self_feats, neigh_sum, W, gamma2, beta2):
    return pl.pallas_call(
        _tc_body,
        out_shape=jax.ShapeDtypeStruct((D, B), jnp.float32),
    )(self_feats, neigh_sum, W, gamma2, beta2)


def kernel(nodes, node_map, neigh_idx, features, W, gamma, beta):
    nidx = neigh_idx.reshape(NW, NCH, C * K)
    self_feats, neigh_sum = _make_sc_gather()(nodes, node_map, nidx, features)
    return _tc_fused(self_feats, neigh_sum, W,
                     gamma.reshape(D, 1), beta.reshape(D, 1))
